# baseline (speedup 1.0000x reference)
import functools

import jax
import jax.numpy as jnp
from jax import lax
from jax.experimental import pallas as pl
from jax.experimental.pallas import tpu as pltpu
from jax.experimental.pallas import tpu_sc as plsc

VS = 1000000
DS = 64
B = 16384
C = 20
N = 20

NC = 2
NS = 16
L = 16
NW = NC * NS
BPW = B // NW
CB = 16
NCHUNK = BPW // CB
SEG = 80
NSEG = CB * C // SEG

_LN2 = 0.6931471805599453
_LOG2_COEF = (
    -3.235854911107787, 7.086135972074948, -7.393883925453409,
    5.6658952659659345, -2.905906931388781, 0.9459083880958161,
    -0.17673384211718712, 0.014440352491874364,
)


def _neg_log(y):
    bits = lax.bitcast_convert_type(y, jnp.int32)
    e = ((bits >> 23) & 0xFF) - 127
    m = lax.bitcast_convert_type((bits & 0x7FFFFF) | 0x3F800000, jnp.float32)
    p = jnp.full((L,), _LOG2_COEF[7], jnp.float32)
    for k in range(6, -1, -1):
        p = p * m + _LOG2_COEF[k]
    return -(e.astype(jnp.float32) + p) * _LN2


def _sc_body(wrd_h, ctx_h, neg_h, iemb_h, oemb_h, out_h,
             widx_v, cidx_v, nidx_v, emb_v, cemb_v, nemb_v,
             pdots_v, ndots_v, acc_v, sem):
    cid = lax.axis_index("c")
    sid = lax.axis_index("s")
    wid = sid * NC + cid

    acc_v[0, :] = jnp.zeros((L,), jnp.float32)
    acc_v[1, :] = jnp.zeros((L,), jnp.float32)

    def chunk_body(j, carry):
        base = wid * BPW + j * CB
        row0 = wid * (BPW * C // SEG) + j * NSEG
        pltpu.sync_copy(wrd_h.at[pl.ds(base, CB)], widx_v)
        pltpu.sync_copy(ctx_h.at[pl.ds(row0, NSEG)], cidx_v)
        pltpu.sync_copy(neg_h.at[pl.ds(row0, NSEG)], nidx_v)
        handles = [pltpu.async_copy(iemb_h.at[widx_v], emb_v, sem)]
        for s in range(NSEG):
            handles.append(pltpu.async_copy(
                oemb_h.at[cidx_v.at[s]], cemb_v.at[pl.ds(s * SEG, SEG)], sem))
            handles.append(pltpu.async_copy(
                oemb_h.at[nidx_v.at[s]], nemb_v.at[pl.ds(s * SEG, SEG)], sem))
        for h in handles:
            h.wait()

        lane15 = lax.iota(jnp.int32, L) == (L - 1)

        def b_body(b, c2):
            e0 = emb_v[b, pl.ds(0, L)]
            e1 = emb_v[b, pl.ds(L, L)]
            e2 = emb_v[b, pl.ds(2 * L, L)]
            e3 = emb_v[b, pl.ds(3 * L, L)]
            for rows_v, dots_v in ((cemb_v, pdots_v), (nemb_v, ndots_v)):
                for c in range(C):
                    r = b * C + c
                    acc = rows_v[r, pl.ds(0, L)] * e0
                    acc = acc + rows_v[r, pl.ds(L, L)] * e1
                    acc = acc + rows_v[r, pl.ds(2 * L, L)] * e2
                    acc = acc + rows_v[r, pl.ds(3 * L, L)] * e3
                    cum = plsc.cumsum(acc)
                    idxv = jnp.full((L,), r, jnp.int32)
                    plsc.store_scatter(dots_v, [idxv], cum, mask=lane15)
            return c2
        lax.fori_loop(0, CB, b_body, 0)

        def g_body(g, c3):
            x = pdots_v[pl.ds(g * L, L)]
            s = 1.0 / (1.0 + jnp.exp(-x))
            y = jnp.clip(s, 1e-6, 1.0 - 1e-6)
            acc_v[0, :] = acc_v[0, :] + _neg_log(y)
            x2 = ndots_v[pl.ds(g * L, L)]
            s2 = 1.0 / (1.0 + jnp.exp(-x2))
            y2 = jnp.clip(1.0 - s2, 1e-6, 1.0 - 1e-6)
            acc_v[1, :] = acc_v[1, :] + _neg_log(y2)
            return c3
        lax.fori_loop(0, CB * C // L, g_body, 0)
        return carry

    lax.fori_loop(0, NCHUNK, chunk_body, 0)
    pltpu.sync_copy(acc_v, out_h.at[wid])


_sc_call = functools.partial(
    pl.kernel,
    out_type=jax.ShapeDtypeStruct((NW, 2, L), jnp.float32),
    mesh=plsc.VectorSubcoreMesh(
        core_axis_name="c", subcore_axis_name="s",
        num_cores=NC, num_subcores=NS),
    compiler_params=pltpu.CompilerParams(
        needs_layout_passes=False, use_tc_tiling_on_sc=False),
    scratch_types=[
        pltpu.VMEM((CB,), jnp.int32),
        pltpu.VMEM((NSEG, SEG), jnp.int32),
        pltpu.VMEM((NSEG, SEG), jnp.int32),
        pltpu.VMEM((CB, DS), jnp.float32),
        pltpu.VMEM((CB * C, DS), jnp.float32),
        pltpu.VMEM((CB * N, DS), jnp.float32),
        pltpu.VMEM((CB * C,), jnp.float32),
        pltpu.VMEM((CB * N,), jnp.float32),
        pltpu.VMEM((2, L), jnp.float32),
        pltpu.SemaphoreType.DMA,
    ],
)(_sc_body)


def kernel(iEmb, oEmb, wrd, ctx, neg):
    wrd_i = wrd.astype(jnp.int32)
    ctx_i = ctx.astype(jnp.int32).reshape(B * C // SEG, SEG)
    neg_i = neg.astype(jnp.int32).reshape(B * N // SEG, SEG)
    parts = _sc_call(wrd_i, ctx_i, neg_i, iEmb, oEmb)
    ploss = parts[:, 0, :].sum() / (B * C)
    nloss = parts[:, 1, :].sum() / (B * N)
    return ploss + nloss
